# baseline (speedup 1.0000x reference)
import functools

import jax
import jax.numpy as jnp
from jax import lax
from jax.experimental import pallas as pl
from jax.experimental.pallas import tpu as pltpu
from jax.experimental.pallas import tpu_sc as plsc

LANES = 16


@functools.cache
def _make_kernel(N, L, D, V, SEG):
    info = plsc.get_sparse_core_info()
    NC, NS = info.num_cores, info.num_subcores
    NW = NC * NS
    rows_per_w = N // NW
    CH = 32
    n_ch = rows_per_w // CH
    ND = D // LANES

    mesh = plsc.VectorSubcoreMesh(core_axis_name="c", subcore_axis_name="s")

    @functools.partial(
        pl.kernel,
        mesh=mesh,
        out_type=jax.ShapeDtypeStruct((N, D), jnp.float32),
        scratch_types=[
            pltpu.VMEM((rows_per_w,), jnp.int32),
            pltpu.VMEM((rows_per_w,), jnp.int32),
            pltpu.VMEM((SEG, D), jnp.float32),
            pltpu.VMEM((CH, D), jnp.float32),
            pltpu.VMEM((CH, D), jnp.float32),
            pltpu.VMEM((CH, D), jnp.float32),
            pltpu.VMEM((CH, D), jnp.float32),
            pltpu.SemaphoreType.DMA,
            pltpu.SemaphoreType.DMA,
            pltpu.SemaphoreType.DMA,
            pltpu.SemaphoreType.DMA,
            pltpu.SemaphoreType.DMA,
            pltpu.SemaphoreType.DMA,
        ],
    )
    def k(tok_hbm, seg_hbm, wte_hbm, wpe_hbm, wse_hbm, out_hbm,
          tok_v, seg_v, wse_v, buf_a, buf_b, pos_a, pos_b,
          sg0, sg1, sp0, sp1, sw0, sw1):
        bufs = (buf_a, buf_b)
        poss = (pos_a, pos_b)
        sgs = (sg0, sg1)
        sps = (sp0, sp1)
        sws = (sw0, sw1)
        wid = lax.axis_index("s") * NC + lax.axis_index("c")
        base = wid * rows_per_w
        l0 = base % L
        pltpu.sync_copy(tok_hbm.at[pl.ds(base, rows_per_w)], tok_v)
        pltpu.sync_copy(seg_hbm.at[pl.ds(base, rows_per_w)], seg_v)
        pltpu.sync_copy(wse_hbm, wse_v)

        def issue(c):
            p = c % 2
            g = pltpu.async_copy(
                wte_hbm.at[tok_v.at[pl.ds(c * CH, CH)]], bufs[p], sgs[p])
            q = pltpu.async_copy(
                wpe_hbm.at[pl.ds(l0 + c * CH, CH)], poss[p], sps[p])
            return g, q

        pend = issue(0)
        wr = [None, None]
        for c in range(n_ch):
            off = c * CH
            p = c % 2
            buf = bufs[p]
            pos_v = poss[p]
            if wr[1 - p] is not None:
                wr[1 - p].wait()
                wr[1 - p] = None
            nxt = issue(c + 1) if c + 1 < n_ch else None
            pend[0].wait()
            pend[1].wait()

            def gbody(g, _):
                row0 = g * LANES
                segs = seg_v[pl.ds(off + row0, LANES)]
                svals = [segs[j] for j in range(LANES)]

                @plsc.parallel_loop(0, ND, unroll=4)
                def dbody(dd):
                    ds_ = pl.ds(dd * LANES, LANES)
                    for j in range(LANES):
                        r = row0 + j
                        plsc.addupdate(
                            buf.at[r, ds_],
                            pos_v[r, ds_] + wse_v[svals[j], ds_])

                return 0

            lax.fori_loop(0, CH // LANES, gbody, 0)
            wr[p] = pltpu.async_copy(
                buf, out_hbm.at[pl.ds(base + off, CH)], sws[p])
            pend = nxt
        for w in wr:
            if w is not None:
                w.wait()

    return k


def kernel(token_ids, segment_ids, wte, wpe, wse):
    B, L = token_ids.shape
    V, D = wte.shape
    N = B * L
    tok = token_ids.reshape(N).astype(jnp.int32)
    seg = segment_ids.reshape(N).astype(jnp.int32)
    k = _make_kernel(N, L, D, V, wse.shape[0])
    out = k(tok, seg, wte, wpe, wse)
    return out.reshape(B, L, D)


# speedup vs baseline: 1.0464x; 1.0464x over previous
import functools

import jax
import jax.numpy as jnp
from jax import lax
from jax.experimental import pallas as pl
from jax.experimental.pallas import tpu as pltpu
from jax.experimental.pallas import tpu_sc as plsc

LANES = 16


@functools.cache
def _make_kernel(N, L, D, V, SEG):
    info = plsc.get_sparse_core_info()
    NC, NS = info.num_cores, info.num_subcores
    NW = NC * NS
    rows_per_w = N // NW
    CH = 32
    n_ch = rows_per_w // CH
    ND = D // LANES

    mesh = plsc.VectorSubcoreMesh(core_axis_name="c", subcore_axis_name="s")

    @functools.partial(
        pl.kernel,
        mesh=mesh,
        out_type=jax.ShapeDtypeStruct((N, D), jnp.float32),
        scratch_types=[
            pltpu.VMEM((rows_per_w,), jnp.int32),
            pltpu.VMEM((rows_per_w,), jnp.int32),
            pltpu.VMEM((SEG, D), jnp.float32),
            pltpu.VMEM((CH, D), jnp.float32),
            pltpu.VMEM((CH, D), jnp.float32),
            pltpu.VMEM((CH, D), jnp.float32),
            pltpu.VMEM((CH, D), jnp.float32),
            pltpu.SemaphoreType.DMA,
            pltpu.SemaphoreType.DMA,
            pltpu.SemaphoreType.DMA,
            pltpu.SemaphoreType.DMA,
            pltpu.SemaphoreType.DMA,
            pltpu.SemaphoreType.DMA,
        ],
    )
    def k(tok_hbm, seg_hbm, wte_hbm, wpe_hbm, wse_hbm, out_hbm,
          tok_v, seg_v, wse_v, buf_a, buf_b, pos_a, pos_b,
          sg0, sg1, sp0, sp1, sw0, sw1):
        bufs = (buf_a, buf_b)
        poss = (pos_a, pos_b)
        sgs = (sg0, sg1)
        sps = (sp0, sp1)
        sws = (sw0, sw1)
        wid = lax.axis_index("s") * NC + lax.axis_index("c")
        base = wid * rows_per_w
        l0 = base % L
        pltpu.sync_copy(tok_hbm.at[pl.ds(base, rows_per_w)], tok_v)
        pltpu.sync_copy(seg_hbm.at[pl.ds(base, rows_per_w)], seg_v)
        pltpu.sync_copy(wse_hbm, wse_v)

        def issue(c):
            p = c % 2
            g = pltpu.async_copy(
                wte_hbm.at[tok_v.at[pl.ds(c * CH, CH)]], bufs[p], sgs[p])
            q = pltpu.async_copy(
                wpe_hbm.at[pl.ds(l0 + c * CH, CH)], poss[p], sps[p])
            return g, q

        pend = issue(0)
        wr = [None, None]
        for c in range(n_ch):
            off = c * CH
            p = c % 2
            buf = bufs[p]
            pos_v = poss[p]
            if wr[1 - p] is not None:
                wr[1 - p].wait()
                wr[1 - p] = None
            nxt = issue(c + 1) if c + 1 < n_ch else None
            pend[0].wait()
            pend[1].wait()

            @plsc.parallel_loop(0, CH // LANES)
            def gbody(g):
                row0 = g * LANES
                segs = seg_v[pl.ds(off + row0, LANES)]
                svals = [segs[j] for j in range(LANES)]

                @plsc.parallel_loop(0, ND, unroll=2)
                def dbody(dd):
                    ds_ = pl.ds(dd * LANES, LANES)
                    for j in range(LANES):
                        r = row0 + j
                        plsc.addupdate(
                            buf.at[r, ds_],
                            pos_v[r, ds_] + wse_v[svals[j], ds_])

            wr[p] = pltpu.async_copy(
                buf, out_hbm.at[pl.ds(base + off, CH)], sws[p])
            pend = nxt
        for w in wr:
            if w is not None:
                w.wait()

    return k


def kernel(token_ids, segment_ids, wte, wpe, wse):
    B, L = token_ids.shape
    V, D = wte.shape
    N = B * L
    tok = token_ids.reshape(N).astype(jnp.int32)
    seg = segment_ids.reshape(N).astype(jnp.int32)
    k = _make_kernel(N, L, D, V, wse.shape[0])
    out = k(tok, seg, wte, wpe, wse)
    return out.reshape(B, L, D)
